# baseline (speedup 1.0000x reference)
import functools

import jax
import jax.numpy as jnp
from jax import lax
from jax.experimental import pallas as pl
from jax.experimental.pallas import tpu as pltpu
from jax.experimental.pallas import tpu_sc as plsc

NUM_EXPERTS = 8
TOP_K = 2
HIDDEN = 1024
FFN = 4096
FBLK = 1024
QUAR = FBLK // 4
TOKENS = 128
LANES = 16


def _logits_kernel(x_ref, gate_ref, out_ref):
    out_ref[...] = jnp.dot(gate_ref[...], x_ref[...].T,
                           preferred_element_type=jnp.float32)


def _route_sc_kernel(logits_hbm, out_hbm, logits_v, wmat_v):
    wid = lax.axis_index("s") * 2 + lax.axis_index("c")

    @pl.when(wid == 0)
    def _():
        pltpu.sync_copy(logits_hbm, logits_v)
        one = jnp.ones((LANES,), jnp.float32)
        zero = jnp.zeros((LANES,), jnp.float32)
        for c in range(TOKENS // LANES):
            sl = pl.ds(c * LANES, LANES)
            le = [logits_v[e, sl] for e in range(NUM_EXPERTS)]
            m = le[0]
            for e in range(1, NUM_EXPERTS):
                m = jnp.maximum(m, le[e])
            p = [jnp.exp(v - m) for v in le]
            m1 = p[0]
            for e in range(1, NUM_EXPERTS):
                m1 = jnp.maximum(m1, p[e])
            seen = zero
            hit1 = []
            for e in range(NUM_EXPERTS):
                eq = jnp.where(p[e] == m1, one, zero)
                h = eq * (one - seen)
                hit1.append(h)
                seen = seen + h
            p2 = [p[e] - 2.0 * hit1[e] for e in range(NUM_EXPERTS)]
            m2 = p2[0]
            for e in range(1, NUM_EXPERTS):
                m2 = jnp.maximum(m2, p2[e])
            seen2 = zero
            hit2 = []
            for e in range(NUM_EXPERTS):
                eq = jnp.where(p2[e] == m2, one, zero)
                h = eq * (one - seen2)
                hit2.append(h)
                seen2 = seen2 + h
            denom = m1 + m2
            r1 = m1 / denom
            r2 = m2 / denom
            for e in range(NUM_EXPERTS):
                wmat_v[e, sl] = hit1[e] * r1 + hit2[e] * r2
        pltpu.sync_copy(wmat_v, out_hbm)


def _route_sc(logits_t):
    mesh = plsc.VectorSubcoreMesh(core_axis_name="c", subcore_axis_name="s")
    kern = functools.partial(
        pl.kernel,
        mesh=mesh,
        out_type=jax.ShapeDtypeStruct((NUM_EXPERTS, TOKENS), jnp.float32),
        scratch_types=[
            pltpu.VMEM((NUM_EXPERTS, TOKENS), jnp.float32),
            pltpu.VMEM((NUM_EXPERTS, TOKENS), jnp.float32),
        ],
    )(_route_sc_kernel)
    return kern(logits_t)


def _moe_kernel(x_ref, wmt_ref, w1a_ref, w1b_ref, w1c_ref, w1d_ref,
                w3a_ref, w3b_ref, w3c_ref, w3d_ref,
                w2a_ref, w2b_ref, w2c_ref, w2d_ref, out_ref, wmat_ref):
    e = pl.program_id(0)
    f = pl.program_id(1)

    @pl.when((e == 0) & (f == 0))
    def _init():
        wmat_ref[...] = wmt_ref[...].T
        out_ref[...] = jnp.zeros_like(out_ref)

    xb = x_ref[...].astype(jnp.bfloat16)
    eoh = (jax.lax.broadcasted_iota(jnp.int32, (NUM_EXPERTS, 1), 0) == e)
    wcol = jnp.dot(wmat_ref[...], eoh.astype(jnp.float32),
                   preferred_element_type=jnp.float32)

    def quarter(w1_ref, w3_ref, w2_ref):
        w1b = w1_ref[0].astype(jnp.bfloat16)
        w3b = w3_ref[0].astype(jnp.bfloat16)
        h1 = jnp.dot(xb, w1b.T, preferred_element_type=jnp.float32)
        h3 = jnp.dot(xb, w3b.T, preferred_element_type=jnp.float32)
        h = (jax.nn.silu(h1) * h3 * wcol).astype(jnp.bfloat16)
        w2b = w2_ref[0].astype(jnp.bfloat16)
        return jnp.dot(h, w2b.T, preferred_element_type=jnp.float32)

    out_ref[...] += (quarter(w1a_ref, w3a_ref, w2a_ref)
                     + quarter(w1b_ref, w3b_ref, w2b_ref)
                     + quarter(w1c_ref, w3c_ref, w2c_ref)
                     + quarter(w1d_ref, w3d_ref, w2d_ref))


@functools.partial(jax.jit, static_argnames=())
def kernel(hidden_states, gate_w, w1, w2, w3):
    b, s, d = hidden_states.shape
    x = hidden_states.reshape(-1, d)
    t = x.shape[0]
    nf = FFN // FBLK

    logits_t = pl.pallas_call(
        _logits_kernel,
        in_specs=[
            pl.BlockSpec((t, HIDDEN), lambda: (0, 0)),
            pl.BlockSpec((NUM_EXPERTS, HIDDEN), lambda: (0, 0)),
        ],
        out_specs=pl.BlockSpec((NUM_EXPERTS, t), lambda: (0, 0)),
        out_shape=jax.ShapeDtypeStruct((NUM_EXPERTS, t), jnp.float32),
    )(x, gate_w)

    wmat_t = _route_sc(logits_t)

    ffn = [pl.BlockSpec((1, QUAR, HIDDEN),
                        (lambda k: (lambda e, f: (e, 4 * f + k, 0)))(k))
           for k in range(4)]
    col = [pl.BlockSpec((1, HIDDEN, QUAR),
                        (lambda k: (lambda e, f: (e, 0, 4 * f + k)))(k))
           for k in range(4)]

    out = pl.pallas_call(
        _moe_kernel,
        grid=(NUM_EXPERTS, nf),
        in_specs=[
            pl.BlockSpec((t, HIDDEN), lambda e, f: (0, 0)),
            pl.BlockSpec((NUM_EXPERTS, t), lambda e, f: (0, 0)),
            *ffn, *ffn, *col,
        ],
        out_specs=pl.BlockSpec((t, HIDDEN), lambda e, f: (0, 0)),
        out_shape=jax.ShapeDtypeStruct((t, HIDDEN), jnp.float32),
        scratch_shapes=[pltpu.VMEM((t, NUM_EXPERTS), jnp.float32)],
    )(x, wmat_t, w1, w1, w1, w1, w3, w3, w3, w3, w2, w2, w2, w2)
    return out.reshape(b, s, d)


# speedup vs baseline: 1.1576x; 1.1576x over previous
import functools

import jax
import jax.numpy as jnp
from jax.experimental import pallas as pl
from jax.experimental.pallas import tpu as pltpu

NUM_EXPERTS = 8
TOP_K = 2
HIDDEN = 1024
FFN = 4096
FBLK = 1024
QUAR = FBLK // 4


def _moe_kernel(x_ref, gate_ref, w1a_ref, w1b_ref, w1c_ref, w1d_ref,
                w3a_ref, w3b_ref, w3c_ref, w3d_ref,
                w2a_ref, w2b_ref, w2c_ref, w2d_ref, out_ref, wmat_ref):
    e = pl.program_id(0)
    f = pl.program_id(1)

    @pl.when((e == 0) & (f == 0))
    def _routing():
        x = x_ref[...]
        logits = jnp.dot(x, gate_ref[...].T, preferred_element_type=jnp.float32)
        p = jax.nn.softmax(logits, axis=-1)
        cols = jax.lax.broadcasted_iota(jnp.int32, p.shape, 1)
        i1 = jnp.argmax(p, axis=-1)
        oh1 = (cols == i1[:, None])
        m1 = jnp.max(p, axis=-1, keepdims=True)
        p2 = jnp.where(oh1, -jnp.inf, p)
        i2 = jnp.argmax(p2, axis=-1)
        oh2 = (cols == i2[:, None])
        m2 = jnp.max(p2, axis=-1, keepdims=True)
        s = m1 + m2
        wmat_ref[...] = oh1 * (m1 / s) + oh2 * (m2 / s)
        out_ref[...] = jnp.zeros_like(out_ref)

    xb = x_ref[...].astype(jnp.bfloat16)
    eoh = (jax.lax.broadcasted_iota(jnp.int32, (NUM_EXPERTS, 1), 0) == e)
    wcol = jnp.dot(wmat_ref[...], eoh.astype(jnp.float32),
                   preferred_element_type=jnp.float32)

    def quarter(w1_ref, w3_ref, w2_ref):
        w1b = w1_ref[0].astype(jnp.bfloat16)
        w3b = w3_ref[0].astype(jnp.bfloat16)
        h1 = jnp.dot(xb, w1b.T, preferred_element_type=jnp.float32)
        h3 = jnp.dot(xb, w3b.T, preferred_element_type=jnp.float32)
        h = (jax.nn.silu(h1) * h3 * wcol).astype(jnp.bfloat16)
        w2b = w2_ref[0].astype(jnp.bfloat16)
        return jnp.dot(h, w2b.T, preferred_element_type=jnp.float32)

    out_ref[...] += (quarter(w1a_ref, w3a_ref, w2a_ref)
                     + quarter(w1b_ref, w3b_ref, w2b_ref)
                     + quarter(w1c_ref, w3c_ref, w2c_ref)
                     + quarter(w1d_ref, w3d_ref, w2d_ref))


@functools.partial(jax.jit, static_argnames=())
def kernel(hidden_states, gate_w, w1, w2, w3):
    b, s, d = hidden_states.shape
    x = hidden_states.reshape(-1, d)
    t = x.shape[0]
    nf = FFN // FBLK

    ffn = [pl.BlockSpec((1, QUAR, HIDDEN),
                        (lambda k: (lambda e, f: (e, 4 * f + k, 0)))(k))
           for k in range(4)]
    col = [pl.BlockSpec((1, HIDDEN, QUAR),
                        (lambda k: (lambda e, f: (e, 0, 4 * f + k)))(k))
           for k in range(4)]

    out = pl.pallas_call(
        _moe_kernel,
        grid=(NUM_EXPERTS, nf),
        in_specs=[
            pl.BlockSpec((t, HIDDEN), lambda e, f: (0, 0)),
            pl.BlockSpec((NUM_EXPERTS, HIDDEN), lambda e, f: (0, 0)),
            *ffn, *ffn, *col,
        ],
        out_specs=pl.BlockSpec((t, HIDDEN), lambda e, f: (0, 0)),
        out_shape=jax.ShapeDtypeStruct((t, HIDDEN), jnp.float32),
        scratch_shapes=[pltpu.VMEM((t, NUM_EXPERTS), jnp.float32)],
    )(x, gate_w, w1, w1, w1, w1, w3, w3, w3, w3, w2, w2, w2, w2)
    return out.reshape(b, s, d)
